# Initial kernel scaffold; baseline (speedup 1.0000x reference)
import jax
import jax.numpy as jnp
from jax.experimental import pallas as pl


def kernel(x, params):
    raise NotImplementedError("write your pallas kernel here")


# speedup vs baseline: 1.6993x; 1.6993x over previous
import jax
import jax.numpy as jnp
from jax.experimental import pallas as pl
from jax.experimental.pallas import tpu as pltpu

NW = 5
EPS = 1e-6
CH = [64, 128, 256, 512]
NDEPTH = [2, 2, 8, 2]
NHEADS = [2, 4, 8, 16]
NTOPK = [1, 4, 16, 25]


def _pick_tn(n):
    for c in (1024, 800, 512, 400, 320, 256, 200, 128, 100, 64, 50, 32, 25, 16, 8):
        if n % c == 0:
            return c
    return n


def _linear(x, w, b, *, g=None, bt=None, act=False, x2=None, res=None,
            split3=False):
    n, k = x.shape
    m = w.shape[1]
    tn = _pick_tn(n)
    nb = n // tn

    args = [x]
    specs = [pl.BlockSpec((tn, k), lambda i: (i, 0))]
    if x2 is not None:
        args.append(x2)
        specs.append(pl.BlockSpec((tn, k), lambda i: (i, 0)))
    if g is not None:
        args += [g.reshape(1, k), bt.reshape(1, k)]
        specs += [pl.BlockSpec((1, k), lambda i: (0, 0))] * 2
    args += [w, b.reshape(1, m)]
    specs += [pl.BlockSpec((k, m), lambda i: (0, 0)),
              pl.BlockSpec((1, m), lambda i: (0, 0))]
    if res is not None:
        args.append(res)
        specs.append(pl.BlockSpec((tn, m), lambda i: (i, 0)))

    def body(*refs):
        it = iter(refs)
        xv = next(it)[...]
        if x2 is not None:
            xv = xv + next(it)[...]
        if g is not None:
            gv = next(it)[...]
            bv = next(it)[...]
            mu = jnp.mean(xv, axis=1, keepdims=True)
            var = jnp.mean((xv - mu) ** 2, axis=1, keepdims=True)
            xv = (xv - mu) / jnp.sqrt(var + EPS) * gv + bv
        wv = next(it)[...]
        bias = next(it)[...]
        y = jnp.dot(xv, wv, preferred_element_type=jnp.float32) + bias
        if act:
            y = jax.nn.gelu(y)
        if res is not None:
            y = y + next(it)[...]
        if split3:
            c = m // 3
            for part in range(3):
                next(it)[...] = y[:, part * c:(part + 1) * c]
        else:
            next(it)[...] = y

    if split3:
        c = m // 3
        out_specs = [pl.BlockSpec((tn, c), lambda i: (i, 0))] * 3
        out_shape = [jax.ShapeDtypeStruct((n, c), jnp.float32)] * 3
    else:
        out_specs = pl.BlockSpec((tn, m), lambda i: (i, 0))
        out_shape = jax.ShapeDtypeStruct((n, m), jnp.float32)
    return pl.pallas_call(
        body,
        grid=(nb,),
        in_specs=specs,
        out_specs=out_specs,
        out_shape=out_shape,
    )(*args)


def _dwconv(x, w, b, res=None):
    H, W, C = x.shape
    xp = jnp.pad(x, ((1, 1), (1, 1), (0, 0)))
    w9 = w.reshape(9, C)
    args = [xp, w9, b.reshape(1, C)]
    if res is not None:
        args.append(res)

    def body(*refs):
        xr, wr, br = refs[0], refs[1], refs[2]
        o_ref = refs[-1]
        wv = wr[...].astype(jnp.bfloat16).astype(jnp.float32)
        xv = xr[...].astype(jnp.bfloat16).astype(jnp.float32)
        acc = jnp.zeros((H, W, C), jnp.float32)
        for i in range(9):
            dy, dx = divmod(i, 3)
            acc = acc + xv[dy:dy + H, dx:dx + W, :] * wv[i, :]
        acc = acc + br[...]
        if res is not None:
            acc = acc + refs[3][...]
        o_ref[...] = acc

    return pl.pallas_call(
        body, out_shape=jax.ShapeDtypeStruct((H, W, C), jnp.float32))(*args)


def _routing(q, k, h, w, C, topk):
    H, W = NW * h, NW * w
    p2 = NW * NW

    def body(q_ref, k_ref, idx_ref):
        def wmean(ref):
            rows = []
            for a in range(NW):
                for b in range(NW):
                    tile = ref[a * h:(a + 1) * h, b * w:(b + 1) * w, :]
                    rows.append(tile.reshape(h * w, C).mean(0, keepdims=True))
            return jnp.concatenate(rows, axis=0)
        qm = wmean(q_ref)
        km = wmean(k_ref)
        r = jax.lax.dot_general(qm, km, (((1,), (1,)), ((), ())),
                                preferred_element_type=jnp.float32)
        iota = jax.lax.broadcasted_iota(jnp.int32, (p2, p2), 1)
        cols = []
        for _ in range(topk):
            mx = jnp.max(r, axis=1, keepdims=True)
            sel = jnp.min(jnp.where(r >= mx, iota, p2), axis=1, keepdims=True)
            cols.append(sel)
            r = jnp.where(iota == sel, -1e30, r)
        idx_ref[...] = jnp.concatenate(cols, axis=1) if topk > 1 else cols[0]

    return pl.pallas_call(
        body,
        grid=(1,),
        in_specs=[pl.BlockSpec((H, W, C), lambda i: (0, 0, 0)),
                  pl.BlockSpec((H, W, C), lambda i: (0, 0, 0))],
        out_specs=pl.BlockSpec((p2, topk), lambda i: (0, 0)),
        out_shape=jax.ShapeDtypeStruct((p2, topk), jnp.int32),
    )(q, k)


def _attn(q, k, v, idx, h, w, C, heads, topk):
    H, W = NW * h, NW * w
    hw = h * w
    hd = C // heads
    scale = hd ** -0.5

    def body(idx_ref, q_ref, k_ref, v_ref, o_ref):
        a = pl.program_id(0)
        b = pl.program_id(1)
        wi = a * NW + b
        q = q_ref[...].reshape(hw, C)
        kc, vc = [], []
        for t in range(topk):
            j = idx_ref[wi, t]
            r0 = (j // NW) * h
            c0 = (j % NW) * w
            kc.append(k_ref[pl.ds(r0, h), pl.ds(c0, w), :].reshape(hw, C))
            vc.append(v_ref[pl.ds(r0, h), pl.ds(c0, w), :].reshape(hw, C))
        kcat = jnp.concatenate(kc, axis=0) if topk > 1 else kc[0]
        vcat = jnp.concatenate(vc, axis=0) if topk > 1 else vc[0]
        outs = []
        for hh in range(heads):
            sl = slice(hh * hd, (hh + 1) * hd)
            s = jax.lax.dot_general(q[:, sl], kcat[:, sl],
                                    (((1,), (1,)), ((), ())),
                                    preferred_element_type=jnp.float32) * scale
            s = s - jnp.max(s, axis=1, keepdims=True)
            e = jnp.exp(s)
            p = e / jnp.sum(e, axis=1, keepdims=True)
            outs.append(jnp.dot(p, vcat[:, sl], preferred_element_type=jnp.float32))
        o_ref[...] = jnp.concatenate(outs, axis=1).reshape(h, w, C)

    grid_spec = pltpu.PrefetchScalarGridSpec(
        num_scalar_prefetch=1,
        grid=(NW, NW),
        in_specs=[
            pl.BlockSpec((h, w, C), lambda a, b, *_: (a, b, 0)),
            pl.BlockSpec((H, W, C), lambda a, b, *_: (0, 0, 0)),
            pl.BlockSpec((H, W, C), lambda a, b, *_: (0, 0, 0)),
        ],
        out_specs=pl.BlockSpec((h, w, C), lambda a, b, *_: (a, b, 0)),
    )
    return pl.pallas_call(
        body, grid_spec=grid_spec,
        out_shape=jax.ShapeDtypeStruct((H, W, C), jnp.float32),
    )(idx, q, k, v)


def _attn_full(q, k, v, H, W, C, heads):
    n = H * W
    hd = C // heads
    scale = hd ** -0.5

    def body(q_ref, k_ref, v_ref, o_ref):
        qv = q_ref[...].reshape(n, C)
        kv = k_ref[...].reshape(n, C)
        vv = v_ref[...].reshape(n, C)
        outs = []
        for hh in range(heads):
            sl = slice(hh * hd, (hh + 1) * hd)
            s = jax.lax.dot_general(qv[:, sl], kv[:, sl],
                                    (((1,), (1,)), ((), ())),
                                    preferred_element_type=jnp.float32) * scale
            s = s - jnp.max(s, axis=1, keepdims=True)
            e = jnp.exp(s)
            p = e / jnp.sum(e, axis=1, keepdims=True)
            outs.append(jnp.dot(p, vv[:, sl], preferred_element_type=jnp.float32))
        o_ref[...] = jnp.concatenate(outs, axis=1).reshape(H, W, C)

    return pl.pallas_call(
        body,
        grid=(1,),
        in_specs=[pl.BlockSpec((H, W, C), lambda i: (0, 0, 0))] * 3,
        out_specs=pl.BlockSpec((H, W, C), lambda i: (0, 0, 0)),
        out_shape=jax.ShapeDtypeStruct((H, W, C), jnp.float32),
    )(q, k, v)


def _im2col_s2(x):
    H, W, Cin = x.shape
    xp = jnp.pad(x, ((0, 1), (0, 1), (0, 0)))
    cols = [xp[dy:dy + H:2, dx:dx + W:2, :]
            for dy in range(3) for dx in range(3)]
    return jnp.concatenate(cols, axis=-1).reshape((H // 2) * (W // 2), 9 * Cin)


def kernel(x, params):
    x = jnp.transpose(x[0], (1, 2, 0))
    feats = []
    for i in range(4):
        d = params['down%d' % i]
        if i == 0:
            y = _linear(_im2col_s2(x), d['w0'].reshape(27, 32), d['b0'],
                        act=True).reshape(320, 320, 32)
            x = _linear(_im2col_s2(y), d['w1'].reshape(288, 64),
                        d['b1']).reshape(160, 160, 64)
        else:
            cin, cout = CH[i - 1], CH[i]
            hn = x.shape[0] // 2
            x = _linear(_im2col_s2(x), d['w0'].reshape(9 * cin, cout),
                        d['b0']).reshape(hn, hn, cout)
        C, heads, tk = CH[i], NHEADS[i], NTOPK[i]
        H = x.shape[0]
        h = H // NW

        def _lnorm(t, g, bt):
            mu = t.mean(-1, keepdims=True)
            var = ((t - mu) ** 2).mean(-1, keepdims=True)
            return (t - mu) / jnp.sqrt(var + EPS) * g + bt

        for j in range(NDEPTH[i]):
            p = params['s%db%d' % (i, j)]
            x = _dwconv(x, p['wpos'], p['bpos'], res=x)
            xf = x.reshape(H * H, C)
            qf, kf, vf = _linear(_lnorm(xf, p['g1'], p['b1']),
                                 p['wqkv'], p['bqkv'], split3=True)
            qi = qf.reshape(H, H, C)
            ki = kf.reshape(H, H, C)
            vi = vf.reshape(H, H, C)
            if tk == NW * NW:
                o = _attn_full(qi, ki, vi, H, H, C, heads)
            else:
                idx = _routing(qi, ki, h, h, C, tk)
                o = _attn(qi, ki, vi, idx, h, h, C, heads, tk)
            lepe = _dwconv(vi, p['wlepe'], p['blepe'])
            xf = _linear(o.reshape(H * H, C), p['wo'], p['bo'],
                         x2=lepe.reshape(H * H, C), res=xf)
            y = _linear(_lnorm(xf, p['g2'], p['b2']), p['w1'], p['bb1'],
                        act=True)
            xf = _linear(y, p['w2'], p['bb2'], res=xf)
            x = xf.reshape(H, H, C)
        if i >= 1:
            feats.append(jnp.transpose(x, (2, 0, 1))[None])
    return tuple(feats)
